# baseline (speedup 1.0000x reference)
import jax
import jax.numpy as jnp
from jax import lax
from jax.experimental import pallas as pl
from jax.experimental.pallas import tpu as pltpu
from jax.experimental.pallas import tpu_sc as plsc

B = 1024
L = 200
D = 64
LP = 208
NW = 32
SPW = B // NW
SPLIT = 104
NGRP = LP // 16
NDC = D // 16


def _body(embs_hbm, sl_hbm, w_hbm, bias_hbm, out_hbm,
          sl_v, emb_v, rows_v0, rows_v1, bias_v0, bias_v1, out_v,
          pidx_v0, pidx_v1, gsem0, gsem1, osem):
    rows_bufs = (rows_v0, rows_v1)
    bias_bufs = (bias_v0, bias_v1)
    pidx_bufs = (pidx_v0, pidx_v1)
    wid = lax.axis_index("s") * 2 + lax.axis_index("c")
    base = wid * SPW

    pltpu.sync_copy(sl_hbm.at[pl.ds(base * L, SPW * L)], sl_v)
    pltpu.sync_copy(embs_hbm.at[pl.ds(base * D, SPW * D)], emb_v)

    gsems = (gsem0, gsem1)

    def gather_cps(j, buf, sem):
        rv, bv, pv = rows_bufs[buf], bias_bufs[buf], pidx_bufs[buf]
        return [
            pltpu.make_async_copy(w_hbm.at[pv.at[pl.ds(0, SPLIT)]],
                                  rv.at[pl.ds(0, SPLIT)], sem),
            pltpu.make_async_copy(w_hbm.at[pv.at[pl.ds(SPLIT, L - SPLIT)]],
                                  rv.at[pl.ds(SPLIT, L - SPLIT)], sem),
            pltpu.make_async_copy(bias_hbm.at[sl_v.at[pl.ds(j * L, SPLIT)]],
                                  bv.at[pl.ds(0, SPLIT)], sem),
            pltpu.make_async_copy(bias_hbm.at[sl_v.at[pl.ds(j * L + SPLIT, L - SPLIT)]],
                                  bv.at[pl.ds(SPLIT, L - SPLIT)], sem),
        ]

    def fire(j, buf, sem):
        pv = pidx_bufs[buf]
        for c in range(NGRP):
            off = pl.multiple_of(j * L, 8) + c * 16
            iv = sl_v[pl.ds(off, 16)]
            pv[pl.ds(c * 16, 16)] = lax.shift_right_logical(iv, 1)
        for cp in gather_cps(j, buf, sem):
            cp.start()

    def drain(j, buf, sem):
        for cp in gather_cps(j, buf, sem):
            cp.wait()

    fire(0, 0, gsems[0])
    fire(1, 1, gsems[1])

    @pl.loop(0, SPW, step=2)
    def step(i0):
        for buf in range(2):
            i = i0 + buf
            sem = gsems[buf]
            drain(i, buf, sem)

            ecs = [emb_v[pl.ds(pl.multiple_of(i * D, 16) + dc * 16, 16)]
                   for dc in range(NDC)]

            rv, bv = rows_bufs[buf], bias_bufs[buf]

            def per_group(g, gcarry):
                row0 = pl.multiple_of(g * 16, 16)
                row_idx = lax.iota(jnp.int32, 16) + row0
                iv = sl_v[pl.ds(pl.multiple_of(i * L, 8) + row0, 16)]
                cb = lax.shift_left(jnp.bitwise_and(iv, 1), 6)
                acc = jnp.zeros((16,), jnp.float32)
                for dc in range(NDC):
                    ec = ecs[dc]
                    for jj in range(16):
                        d = dc * 16 + jj
                        acc = acc + plsc.load_gather(rv, [row_idx, cb + d]) * ec[jj]
                out_v[pl.ds(pl.multiple_of(i * LP, 16) + row0, 16)] = (
                    acc + bv[pl.ds(row0, 16)])
                return gcarry

            lax.fori_loop(0, NGRP, per_group, None)

            @pl.when(i + 2 < SPW)
            def _prefetch():
                fire(i + 2, buf, sem)

            pltpu.make_async_copy(out_v.at[pl.ds(i * LP, L)],
                                  out_hbm.at[pl.ds((base + i) * L, L)],
                                  osem).start()

    @pl.loop(0, SPW)
    def drain_out(j):
        pltpu.make_async_copy(out_v.at[pl.ds(j * LP, L)],
                              out_hbm.at[pl.ds((base + j) * L, L)],
                              osem).wait()


@jax.jit
def kernel(embs, shortlist, weight, bias):
    sl1 = shortlist.astype(jnp.int32).reshape(-1)
    e1 = embs.reshape(-1)
    mesh = plsc.VectorSubcoreMesh(core_axis_name="c", subcore_axis_name="s")
    run = pl.kernel(
        _body,
        out_type=jax.ShapeDtypeStruct((B * L,), jnp.float32),
        mesh=mesh,
        compiler_params=pltpu.CompilerParams(
            needs_layout_passes=False, use_tc_tiling_on_sc=True),
        scratch_types=[
            pltpu.VMEM((SPW * L,), jnp.int32),
            pltpu.VMEM((SPW * D,), jnp.float32),
            pltpu.VMEM((LP, 2 * D), jnp.float32),
            pltpu.VMEM((LP, 2 * D), jnp.float32),
            pltpu.VMEM((LP,), jnp.float32),
            pltpu.VMEM((LP,), jnp.float32),
            pltpu.VMEM((SPW * LP,), jnp.float32),
            pltpu.VMEM((LP,), jnp.int32),
            pltpu.VMEM((LP,), jnp.int32),
            pltpu.SemaphoreType.DMA,
            pltpu.SemaphoreType.DMA,
            pltpu.SemaphoreType.DMA,
        ],
    )
    w2 = weight.reshape(B // 2 * 0 + 500000, 128)
    out1 = run(e1, sl1, w2, bias)
    return out1.reshape(B, L)


# speedup vs baseline: 1.1655x; 1.1655x over previous
import jax
import jax.numpy as jnp
from jax import lax
from jax.experimental import pallas as pl
from jax.experimental.pallas import tpu as pltpu
from jax.experimental.pallas import tpu_sc as plsc

B = 1024
L = 200
D = 64
LP = 208
NW = 32
SPW = B // NW
SPLIT = 104
NGRP = LP // 16
NDC = D // 16


def _body(embs_hbm, sl_hbm, w_hbm, bias_hbm, out_hbm,
          sl_v, emb_v, rows_v, tbuf_v, bias_v, out_v,
          gsem0, gsem1, osem):
    wid = lax.axis_index("s") * 2 + lax.axis_index("c")
    base = wid * SPW

    pltpu.sync_copy(sl_hbm.at[pl.ds(base, SPW)], sl_v)
    pltpu.sync_copy(embs_hbm.at[pl.ds(base, SPW)], emb_v)

    gsems = (gsem0, gsem1)

    def gather_cps(j, buf, sem):
        return [
            pltpu.make_async_copy(w_hbm.at[sl_v.at[j, pl.ds(0, SPLIT)]],
                                  rows_v.at[buf, pl.ds(0, SPLIT)], sem),
            pltpu.make_async_copy(w_hbm.at[sl_v.at[j, pl.ds(SPLIT, L - SPLIT)]],
                                  rows_v.at[buf, pl.ds(SPLIT, L - SPLIT)], sem),
            pltpu.make_async_copy(bias_hbm.at[sl_v.at[j, pl.ds(0, SPLIT)]],
                                  bias_v.at[buf, pl.ds(0, SPLIT)], sem),
            pltpu.make_async_copy(bias_hbm.at[sl_v.at[j, pl.ds(SPLIT, L - SPLIT)]],
                                  bias_v.at[buf, pl.ds(SPLIT, L - SPLIT)], sem),
        ]

    def fire(j, buf, sem):
        for cp in gather_cps(j, buf, sem):
            cp.start()

    def drain(j, buf, sem):
        for cp in gather_cps(j, buf, sem):
            cp.wait()

    fire(0, 0, gsems[0])
    fire(1, 1, gsems[1])

    @pl.loop(0, SPW, step=2)
    def step(i0):
        for buf in range(2):
            i = i0 + buf
            sem = gsems[buf]
            drain(i, buf, sem)

            ecs = [emb_v[i, pl.ds(dc * 16, 16)] for dc in range(NDC)]

            def per_group(g, gcarry):
                row0 = pl.multiple_of(g * 16, 16)
                for j in range(16):
                    p = rows_v[buf, row0 + j, pl.ds(0, 16)] * ecs[0]
                    for dc in range(1, NDC):
                        p = p + rows_v[buf, row0 + j, pl.ds(dc * 16, 16)] * ecs[dc]
                    tbuf_v[pl.ds(j * 16, 16)] = p
                tbase = lax.iota(jnp.int32, 16) * 16
                acc = plsc.load_gather(tbuf_v, [tbase])
                for k in range(1, 16):
                    acc = acc + plsc.load_gather(tbuf_v, [tbase + k])
                out_v[i, pl.ds(row0, 16)] = acc + bias_v[buf, pl.ds(row0, 16)]
                return gcarry

            lax.fori_loop(0, NGRP, per_group, None)

            @pl.when(i + 2 < SPW)
            def _prefetch():
                fire(i + 2, buf, sem)

            pltpu.make_async_copy(out_v.at[i, pl.ds(0, L)],
                                  out_hbm.at[base + i], osem).start()

    @pl.loop(0, SPW)
    def drain_out(j):
        pltpu.make_async_copy(out_v.at[j, pl.ds(0, L)],
                              out_hbm.at[base + j], osem).wait()


@jax.jit
def kernel(embs, shortlist, weight, bias):
    shortlist = shortlist.astype(jnp.int32)
    mesh = plsc.VectorSubcoreMesh(core_axis_name="c", subcore_axis_name="s")
    run = pl.kernel(
        _body,
        out_type=jax.ShapeDtypeStruct((B, L), jnp.float32),
        mesh=mesh,
        compiler_params=pltpu.CompilerParams(
            needs_layout_passes=False, use_tc_tiling_on_sc=False),
        scratch_types=[
            pltpu.VMEM((SPW, L), jnp.int32),
            pltpu.VMEM((SPW, D), jnp.float32),
            pltpu.VMEM((2, LP, D), jnp.float32),
            pltpu.VMEM((256,), jnp.float32),
            pltpu.VMEM((2, LP), jnp.float32),
            pltpu.VMEM((SPW, LP), jnp.float32),
            pltpu.SemaphoreType.DMA,
            pltpu.SemaphoreType.DMA,
            pltpu.SemaphoreType.DMA,
        ],
    )
    return run(embs, shortlist, weight, bias)


# speedup vs baseline: 1.2820x; 1.1000x over previous
import jax
import jax.numpy as jnp
from jax import lax
from jax.experimental import pallas as pl
from jax.experimental.pallas import tpu as pltpu
from jax.experimental.pallas import tpu_sc as plsc

B = 1024
L = 200
D = 64
LP = 208
NW = 32
SPW = B // NW
SPLIT = 104
NGRP = LP // 16
NDC = D // 16


def _body(embs_hbm, sl_hbm, w_hbm, bias_hbm, out_hbm,
          sl_v, emb_v, rows_v, tbuf_v, bias_v, out_v,
          gsem0, gsem1, osem):
    wid = lax.axis_index("s") * 2 + lax.axis_index("c")
    base = wid * SPW

    pltpu.sync_copy(sl_hbm.at[pl.ds(base, SPW)], sl_v)
    pltpu.sync_copy(embs_hbm.at[pl.ds(base, SPW)], emb_v)

    gsems = (gsem0, gsem1)

    def gather_cps(j, buf, sem):
        return [
            pltpu.make_async_copy(w_hbm.at[sl_v.at[j, pl.ds(0, SPLIT)]],
                                  rows_v.at[buf, pl.ds(0, SPLIT)], sem),
            pltpu.make_async_copy(w_hbm.at[sl_v.at[j, pl.ds(SPLIT, L - SPLIT)]],
                                  rows_v.at[buf, pl.ds(SPLIT, L - SPLIT)], sem),
            pltpu.make_async_copy(bias_hbm.at[sl_v.at[j, pl.ds(0, SPLIT)]],
                                  bias_v.at[buf, pl.ds(0, SPLIT)], sem),
            pltpu.make_async_copy(bias_hbm.at[sl_v.at[j, pl.ds(SPLIT, L - SPLIT)]],
                                  bias_v.at[buf, pl.ds(SPLIT, L - SPLIT)], sem),
        ]

    def fire(j, buf, sem):
        for cp in gather_cps(j, buf, sem):
            cp.start()

    def drain(j, buf, sem):
        for cp in gather_cps(j, buf, sem):
            cp.wait()

    fire(0, 0, gsems[0])
    fire(1, 1, gsems[1])

    @pl.loop(0, SPW, step=2)
    def step(i0):
        for buf in range(2):
            i = i0 + buf
            sem = gsems[buf]
            drain(i, buf, sem)

            ecs = [emb_v[i, pl.ds(dc * 16, 16)] for dc in range(NDC)]

            def per_group(g, gcarry):
                row0 = pl.multiple_of(g * 16, 16)
                for j in range(16):
                    p = rows_v[buf, row0 + j, pl.ds(0, 16)] * ecs[0]
                    for dc in range(1, NDC):
                        p = p + rows_v[buf, row0 + j, pl.ds(dc * 16, 16)] * ecs[dc]
                    tbuf_v[pl.ds(j * 16, 16)] = p
                tbase = lax.iota(jnp.int32, 16) * 16
                acc = plsc.load_gather(tbuf_v, [tbase])
                for k in range(1, 16):
                    acc = acc + plsc.load_gather(tbuf_v, [tbase + k])
                out_v[i, pl.ds(row0, 16)] = acc + bias_v[buf, pl.ds(row0, 16)]
                return gcarry

            lax.fori_loop(0, NGRP, per_group, None)

            @pl.when(i + 2 < SPW)
            def _prefetch():
                fire(i + 2, buf, sem)

            pltpu.make_async_copy(out_v.at[i, pl.ds(0, L)],
                                  out_hbm.at[base + i], osem).start()

    @pl.loop(0, SPW)
    def drain_out(j):
        pltpu.make_async_copy(out_v.at[j, pl.ds(0, L)],
                              out_hbm.at[base + j], osem).wait()


@jax.jit
def kernel(embs, shortlist, weight, bias):
    shortlist = shortlist.astype(jnp.int32)
    weight = jnp.pad(weight, ((0, 0), (0, D)))
    mesh = plsc.VectorSubcoreMesh(core_axis_name="c", subcore_axis_name="s")
    run = pl.kernel(
        _body,
        out_type=jax.ShapeDtypeStruct((B, L), jnp.float32),
        mesh=mesh,
        compiler_params=pltpu.CompilerParams(
            needs_layout_passes=False, use_tc_tiling_on_sc=False),
        scratch_types=[
            pltpu.VMEM((SPW, L), jnp.int32),
            pltpu.VMEM((SPW, D), jnp.float32),
            pltpu.VMEM((2, LP, 2 * D), jnp.float32),
            pltpu.VMEM((256,), jnp.float32),
            pltpu.VMEM((2, LP), jnp.float32),
            pltpu.VMEM((SPW, LP), jnp.float32),
            pltpu.SemaphoreType.DMA,
            pltpu.SemaphoreType.DMA,
            pltpu.SemaphoreType.DMA,
        ],
    )
    return run(embs, shortlist, weight, bias)


# speedup vs baseline: 1.2828x; 1.0006x over previous
import jax
import jax.numpy as jnp
from jax import lax
from jax.experimental import pallas as pl
from jax.experimental.pallas import tpu as pltpu
from jax.experimental.pallas import tpu_sc as plsc

B = 1024
NUMX_PAD = 1000000
L = 200
D = 64
LP = 208
NW = 32
SPW = B // NW
SPLIT = 104
NGRP = LP // 16
NDC = D // 16


def _body(embs_hbm, sl_hbm, w_hbm, bias_hbm, out_hbm,
          sl_v, emb_v, rows_v, tbuf_v, bias_v, out_v,
          pidx_v0, pidx_v1, gsem0, gsem1, osem):
    pidx_bufs = (pidx_v0, pidx_v1)
    wid = lax.axis_index("s") * 2 + lax.axis_index("c")
    base = wid * SPW

    pltpu.sync_copy(sl_hbm.at[pl.ds(base, SPW)], sl_v)
    pltpu.sync_copy(embs_hbm.at[pl.ds(base, SPW)], emb_v)

    gsems = (gsem0, gsem1)

    def gather_cps(j, buf, sem):
        pv = pidx_bufs[buf]
        return [
            pltpu.make_async_copy(w_hbm.at[pv.at[pl.ds(0, SPLIT)]],
                                  rows_v.at[buf, pl.ds(0, SPLIT)], sem),
            pltpu.make_async_copy(w_hbm.at[pv.at[pl.ds(SPLIT, L - SPLIT)]],
                                  rows_v.at[buf, pl.ds(SPLIT, L - SPLIT)], sem),
            pltpu.make_async_copy(bias_hbm.at[sl_v.at[j, pl.ds(0, SPLIT)]],
                                  bias_v.at[buf, pl.ds(0, SPLIT)], sem),
            pltpu.make_async_copy(bias_hbm.at[sl_v.at[j, pl.ds(SPLIT, L - SPLIT)]],
                                  bias_v.at[buf, pl.ds(SPLIT, L - SPLIT)], sem),
        ]

    def fire(j, buf, sem):
        pv = pidx_bufs[buf]
        for off in list(range(0, L - 16, 16)) + [L - 16]:
            iv = sl_v[j, pl.ds(off, 16)]
            pv[pl.ds(off, 16)] = lax.shift_left(iv, 1)
        for cp in gather_cps(j, buf, sem):
            cp.start()

    def drain(j, buf, sem):
        for cp in gather_cps(j, buf, sem):
            cp.wait()

    fire(0, 0, gsems[0])
    fire(1, 1, gsems[1])

    @pl.loop(0, SPW, step=2)
    def step(i0):
        for buf in range(2):
            i = i0 + buf
            sem = gsems[buf]
            drain(i, buf, sem)

            ecs = [emb_v[i, pl.ds(dc * 16, 16)] for dc in range(NDC)]

            def per_group(g, gcarry):
                row0 = pl.multiple_of(g * 16, 16)
                for j in range(16):
                    p = rows_v[buf, row0 + j, pl.ds(0, 16)] * ecs[0]
                    for dc in range(1, NDC):
                        p = p + rows_v[buf, row0 + j, pl.ds(dc * 16, 16)] * ecs[dc]
                    tbuf_v[pl.ds(j * 16, 16)] = p
                tbase = lax.iota(jnp.int32, 16) * 16
                acc = plsc.load_gather(tbuf_v, [tbase])
                for k in range(1, 16):
                    acc = acc + plsc.load_gather(tbuf_v, [tbase + k])
                out_v[i, pl.ds(row0, 16)] = acc + bias_v[buf, pl.ds(row0, 16)]
                return gcarry

            lax.fori_loop(0, NGRP, per_group, None)

            @pl.when(i + 2 < SPW)
            def _prefetch():
                fire(i + 2, buf, sem)

            pltpu.make_async_copy(out_v.at[i, pl.ds(0, L)],
                                  out_hbm.at[base + i], osem).start()

    @pl.loop(0, SPW)
    def drain_out(j):
        pltpu.make_async_copy(out_v.at[j, pl.ds(0, L)],
                              out_hbm.at[base + j], osem).wait()


@jax.jit
def kernel(embs, shortlist, weight, bias):
    shortlist = shortlist.astype(jnp.int32)
    weight = jnp.pad(weight, ((0, 0), (0, D))).reshape(2 * NUMX_PAD, D)
    mesh = plsc.VectorSubcoreMesh(core_axis_name="c", subcore_axis_name="s")
    run = pl.kernel(
        _body,
        out_type=jax.ShapeDtypeStruct((B, L), jnp.float32),
        mesh=mesh,
        compiler_params=pltpu.CompilerParams(
            needs_layout_passes=False, use_tc_tiling_on_sc=False),
        scratch_types=[
            pltpu.VMEM((SPW, L), jnp.int32),
            pltpu.VMEM((SPW, D), jnp.float32),
            pltpu.VMEM((2, LP, D), jnp.float32),
            pltpu.VMEM((256,), jnp.float32),
            pltpu.VMEM((2, LP), jnp.float32),
            pltpu.VMEM((SPW, LP), jnp.float32),
            pltpu.VMEM((LP,), jnp.int32),
            pltpu.VMEM((LP,), jnp.int32),
            pltpu.SemaphoreType.DMA,
            pltpu.SemaphoreType.DMA,
            pltpu.SemaphoreType.DMA,
        ],
    )
    return run(embs, shortlist, weight, bias)
